# Initial kernel scaffold; baseline (speedup 1.0000x reference)
import jax
import jax.numpy as jnp
from jax.experimental import pallas as pl


def kernel(x, edge_index, params):
    raise NotImplementedError("write your pallas kernel here")


# speedup vs baseline: 4.3140x; 4.3140x over previous
import functools

import jax
import jax.numpy as jnp
from jax import lax
from jax.experimental import pallas as pl
from jax.experimental.pallas import tpu as pltpu
from jax.experimental.pallas import tpu_sc as plsc

N = 10000
E = 320000
D_IN = 128
HID = 64
D_OUT = 128

NC = 2
NS = 16
NW = NC * NS
L = 16
EW = E // NW
C = 80
K = EW // C
N2 = 10240
NT = N2 // NS

_mesh = plsc.VectorSubcoreMesh(core_axis_name="c", subcore_axis_name="s")


def _f32(*shape):
    return jax.ShapeDtypeStruct(shape, jnp.float32)


def _gather_add_body(a_hbm, b_hbm, dst_hbm, src_hbm, p_hbm,
                     dst_v, src_v, ra, rb, sem):
    wid = lax.axis_index("s") * NC + lax.axis_index("c")
    base = wid * EW
    pltpu.sync_copy(dst_hbm.at[wid], dst_v)
    pltpu.sync_copy(src_hbm.at[wid], src_v)

    def chunk(j, carry):
        pltpu.async_copy(a_hbm.at[dst_v.at[j]], ra, sem).wait()
        pltpu.async_copy(b_hbm.at[src_v.at[j]], rb, sem).wait()

        def addrow(r, c2):
            for u in range(HID // L):
                sl = pl.ds(u * L, L)
                ra[r, sl] = ra[r, sl] + rb[r, sl]
            return c2

        lax.fori_loop(0, C, addrow, 0)
        pltpu.sync_copy(ra, p_hbm.at[pl.ds(base + j * C, C)])
        return carry

    lax.fori_loop(0, K, chunk, 0)


def _gather_add(a, b, dst3, src3):
    fn = pl.kernel(
        _gather_add_body,
        out_type=[_f32(E, HID)],
        mesh=_mesh,
        compiler_params=pltpu.CompilerParams(use_tc_tiling_on_sc=False),
        scratch_types=[
            pltpu.VMEM((K, C), jnp.int32),
            pltpu.VMEM((K, C), jnp.int32),
            pltpu.VMEM((C, HID), jnp.float32),
            pltpu.VMEM((C, HID), jnp.float32),
            pltpu.SemaphoreType.DMA,
        ],
    )
    return fn(a, b, dst3, src3)[0]


def _scatter_body(with_deg, m_hbm, dst_hbm, out_hbm, deg_hbm,
                  dst_v, rows, zbuf, onesv, zdeg, acc_sh, deg_sh):
    cid = lax.axis_index("c")
    sid = lax.axis_index("s")
    wid = sid * NC + cid
    base = wid * EW

    zero16 = jnp.zeros((L,), jnp.float32)

    def zrow(r, c2):
        for u in range(HID // L):
            zbuf[r, pl.ds(u * L, L)] = zero16
        return c2

    lax.fori_loop(0, NT, zrow, 0)
    pltpu.sync_copy(zbuf, acc_sh.at[pl.ds(sid * NT, NT)])

    if with_deg:
        one16 = jnp.ones((L,), jnp.float32)

        def orow(r, c2):
            onesv[r, :] = one16
            return c2

        lax.fori_loop(0, C, orow, 0)

        def zdrow(r, c2):
            zdeg[r, :] = zero16
            return c2

        lax.fori_loop(0, NT, zdrow, 0)
        pltpu.sync_copy(zdeg, deg_sh.at[pl.ds(sid * NT, NT)])

    plsc.subcore_barrier()

    pltpu.sync_copy(dst_hbm.at[wid], dst_v)

    def chunk(j, carry):
        pltpu.sync_copy(m_hbm.at[pl.ds(base + j * C, C)], rows)
        pltpu.sync_copy(rows, acc_sh.at[dst_v.at[j]], add=True)
        if with_deg:
            pltpu.sync_copy(onesv, deg_sh.at[dst_v.at[j]], add=True)
        return carry

    lax.fori_loop(0, K, chunk, 0)

    plsc.subcore_barrier()

    sl = pl.ds(sid * NT, NT)
    pltpu.sync_copy(acc_sh.at[sl], zbuf)
    pltpu.sync_copy(zbuf, out_hbm.at[cid, sl])
    if with_deg:
        pltpu.sync_copy(deg_sh.at[sl], zdeg)
        pltpu.sync_copy(zdeg, deg_hbm.at[cid, sl])


def _scatter(m, dst3, with_deg):
    fn = pl.kernel(
        functools.partial(_scatter_body, with_deg),
        out_type=[_f32(NC, N2, HID), _f32(NC, N2, L)],
        mesh=_mesh,
        compiler_params=pltpu.CompilerParams(use_tc_tiling_on_sc=False),
        scratch_types=[
            pltpu.VMEM((K, C), jnp.int32),
            pltpu.VMEM((C, HID), jnp.float32),
            pltpu.VMEM((NT, HID), jnp.float32),
            pltpu.VMEM((C, L), jnp.float32),
            pltpu.VMEM((NT, L), jnp.float32),
            pltpu.VMEM_SHARED((N2, HID), jnp.float32),
            pltpu.VMEM_SHARED((N2, L), jnp.float32),
        ],
    )
    return fn(m, dst3)


def _proj_body(x_ref, wd_ref, wb_ref, b_ref, a_ref, bproj_ref):
    xv = x_ref[...]
    a_ref[...] = jnp.dot(xv, wd_ref[...],
                         preferred_element_type=jnp.float32) + b_ref[...]
    bproj_ref[...] = jnp.dot(xv, wb_ref[...],
                             preferred_element_type=jnp.float32)


def _proj(x, wd, wb, b):
    d = x.shape[1]
    bn = 2000
    grid = (N // bn,)
    return pl.pallas_call(
        _proj_body,
        grid=grid,
        in_specs=[
            pl.BlockSpec((bn, d), lambda i: (i, 0)),
            pl.BlockSpec((d, HID), lambda i: (0, 0)),
            pl.BlockSpec((d, HID), lambda i: (0, 0)),
            pl.BlockSpec((1, HID), lambda i: (0, 0)),
        ],
        out_specs=[
            pl.BlockSpec((bn, HID), lambda i: (i, 0)),
            pl.BlockSpec((bn, HID), lambda i: (i, 0)),
        ],
        out_shape=[_f32(N, HID), _f32(N, HID)],
    )(x, wd, wb, b.reshape(1, HID))


def _mlp_body(p_ref, w2_ref, b2_ref, w3_ref, b3_ref, m_ref):
    h = jnp.maximum(p_ref[...], 0.0)
    h = jnp.dot(h, w2_ref[...], preferred_element_type=jnp.float32) + b2_ref[...]
    h = jnp.maximum(h, 0.0)
    h = jnp.dot(h, w3_ref[...], preferred_element_type=jnp.float32) + b3_ref[...]
    m_ref[...] = jnp.maximum(h, 0.0)


def _edge_mlp(p, w2d, b2d, w3d, b3d):
    e2 = E // 2
    be = 4000
    p2 = p.reshape(e2, 2 * HID)
    out = pl.pallas_call(
        _mlp_body,
        grid=(e2 // be,),
        in_specs=[
            pl.BlockSpec((be, 2 * HID), lambda i: (i, 0)),
            pl.BlockSpec((2 * HID, 2 * HID), lambda i: (0, 0)),
            pl.BlockSpec((1, 2 * HID), lambda i: (0, 0)),
            pl.BlockSpec((2 * HID, 2 * HID), lambda i: (0, 0)),
            pl.BlockSpec((1, 2 * HID), lambda i: (0, 0)),
        ],
        out_specs=pl.BlockSpec((be, 2 * HID), lambda i: (i, 0)),
        out_shape=_f32(e2, 2 * HID),
    )(p2, w2d, b2d.reshape(1, 2 * HID), w3d, b3d.reshape(1, 2 * HID))
    return out.reshape(E, HID)


def _node1_body(s_ref, deg_ref, w4_ref, b4_ref, vd_ref, vb_ref, bv_ref,
                a_ref, b_ref):
    s = s_ref[0] + s_ref[1]
    deg = deg_ref[0, :, 0:1] + deg_ref[1, :, 0:1]
    agg = jnp.dot(s, w4_ref[...], preferred_element_type=jnp.float32)
    y = jnp.maximum(agg + deg * b4_ref[...], 0.0)
    a_ref[...] = jnp.dot(y, vd_ref[...],
                         preferred_element_type=jnp.float32) + bv_ref[...]
    b_ref[...] = jnp.dot(y, vb_ref[...], preferred_element_type=jnp.float32)


def _node1(s, deg, w4, b4, vd, vb, bv):
    bn = 2000
    return pl.pallas_call(
        _node1_body,
        grid=(N // bn,),
        in_specs=[
            pl.BlockSpec((NC, bn, HID), lambda i: (0, i, 0)),
            pl.BlockSpec((NC, bn, L), lambda i: (0, i, 0)),
            pl.BlockSpec((HID, HID), lambda i: (0, 0)),
            pl.BlockSpec((1, HID), lambda i: (0, 0)),
            pl.BlockSpec((HID, HID), lambda i: (0, 0)),
            pl.BlockSpec((HID, HID), lambda i: (0, 0)),
            pl.BlockSpec((1, HID), lambda i: (0, 0)),
        ],
        out_specs=[
            pl.BlockSpec((bn, HID), lambda i: (i, 0)),
            pl.BlockSpec((bn, HID), lambda i: (i, 0)),
        ],
        out_shape=[_f32(N, HID), _f32(N, HID)],
    )(s, deg, w4, b4.reshape(1, HID), vd, vb, bv.reshape(1, HID))


def _final_body(s_ref, deg_ref, w4_ref, b4_ref, o_ref):
    s = s_ref[0] + s_ref[1]
    deg = deg_ref[0, :, 0:1] + deg_ref[1, :, 0:1]
    agg = jnp.dot(s, w4_ref[...], preferred_element_type=jnp.float32)
    o_ref[...] = agg + deg * b4_ref[...]


def _final(s, deg, w4, b4):
    bn = 2000
    return pl.pallas_call(
        _final_body,
        grid=(N // bn,),
        in_specs=[
            pl.BlockSpec((NC, bn, HID), lambda i: (0, i, 0)),
            pl.BlockSpec((NC, bn, L), lambda i: (0, i, 0)),
            pl.BlockSpec((HID, D_OUT), lambda i: (0, 0)),
            pl.BlockSpec((1, D_OUT), lambda i: (0, 0)),
        ],
        out_specs=pl.BlockSpec((bn, D_OUT), lambda i: (i, 0)),
        out_shape=_f32(N, D_OUT),
    )(s, deg, w4, b4.reshape(1, D_OUT))


def _blockdiag(w):
    z = jnp.zeros_like(w)
    return jnp.concatenate(
        [jnp.concatenate([w, z], axis=1), jnp.concatenate([z, w], axis=1)],
        axis=0)


def kernel(x, edge_index, params):
    p1 = params["conv1"]
    p2 = params["conv2"]
    (w1, b1), (w2, b2), (w3, b3), (w4, b4) = p1
    (v1, c1), (v2, c2), (v3, c3), (v4, c4) = p2

    w1a, w1b = w1[:D_IN], w1[D_IN:]
    wd1 = w1a - w1b
    v1a, v1b = v1[:HID], v1[HID:]
    vd1 = v1a - v1b

    w2d, w3d = _blockdiag(w2), _blockdiag(w3)
    v2d, v3d = _blockdiag(v2), _blockdiag(v3)
    b2d = jnp.concatenate([b2, b2])
    b3d = jnp.concatenate([b3, b3])
    c2d = jnp.concatenate([c2, c2])
    c3d = jnp.concatenate([c3, c3])

    src3 = edge_index[0].reshape(NW, K, C)
    dst3 = edge_index[1].reshape(NW, K, C)

    a1, bp1 = _proj(x, wd1, w1b, b1)
    pre1 = _gather_add(a1, bp1, dst3, src3)
    m1 = _edge_mlp(pre1, w2d, b2d, w3d, b3d)
    s1, deg = _scatter(m1, dst3, with_deg=True)

    a2, bp2 = _node1(s1, deg, w4, b4, vd1, v1b, c1)
    pre2 = _gather_add(a2, bp2, dst3, src3)
    m2 = _edge_mlp(pre2, v2d, c2d, v3d, c3d)
    s2, _ = _scatter(m2, dst3, with_deg=False)

    return _final(s2, deg, v4, c4)


# speedup vs baseline: 7.4735x; 1.7324x over previous
import functools

import jax
import jax.numpy as jnp
from jax import lax
from jax.experimental import pallas as pl
from jax.experimental.pallas import tpu as pltpu
from jax.experimental.pallas import tpu_sc as plsc

N = 10000
E = 320000
D_IN = 128
HID = 64
D_OUT = 128

NC = 2
NS = 16
NW = NC * NS
L = 16
EW = E // NW
C = 80
K = EW // C
N2 = 10240
NT = N2 // NS

_mesh = plsc.VectorSubcoreMesh(core_axis_name="c", subcore_axis_name="s")


def _f32(*shape):
    return jax.ShapeDtypeStruct(shape, jnp.float32)


def _gather_add_body(a_hbm, b_hbm, dst_hbm, src_hbm, p_hbm,
                     dst_v, src_v, ra0, rb0, ra1, rb1, sem0, sem1):
    wid = lax.axis_index("s") * NC + lax.axis_index("c")
    base = wid * EW
    pltpu.sync_copy(dst_hbm.at[wid], dst_v)
    pltpu.sync_copy(src_hbm.at[wid], src_v)

    bufs = ((ra0, rb0, sem0), (ra1, rb1, sem1))

    def start(j, ra, rb, sem):
        pltpu.async_copy(a_hbm.at[dst_v.at[j]], ra, sem)
        pltpu.async_copy(b_hbm.at[src_v.at[j]], rb, sem)

    def drain(ra, rb, sem):
        pltpu.make_async_copy(a_hbm.at[pl.ds(0, C)], ra, sem).wait()
        pltpu.make_async_copy(b_hbm.at[pl.ds(0, C)], rb, sem).wait()

    def process(j, ra, rb, sem):
        drain(ra, rb, sem)

        @plsc.parallel_loop(0, C, 1, unroll=4)
        def _(r):
            for u in range(HID // L):
                sl = pl.ds(u * L, L)
                ra[r, sl] = ra[r, sl] + rb[r, sl]

        pltpu.sync_copy(ra, p_hbm.at[pl.ds(base + j * C, C)])

    start(0, *bufs[0])
    start(1, *bufs[1])

    @pl.loop(0, K - 1, step=2)
    def _(g):
        for b in range(2):
            j = g + b
            ra, rb, sem = bufs[b]
            process(j, ra, rb, sem)

            @pl.when(j + 2 < K)
            def _():
                start(j + 2, ra, rb, sem)

    process(K - 1, *bufs[(K - 1) % 2])


def _gather_add(a, b, dst3, src3):
    fn = pl.kernel(
        _gather_add_body,
        out_type=[_f32(E, HID)],
        mesh=_mesh,
        compiler_params=pltpu.CompilerParams(use_tc_tiling_on_sc=False),
        scratch_types=[
            pltpu.VMEM((K, C), jnp.int32),
            pltpu.VMEM((K, C), jnp.int32),
            pltpu.VMEM((C, HID), jnp.float32),
            pltpu.VMEM((C, HID), jnp.float32),
            pltpu.VMEM((C, HID), jnp.float32),
            pltpu.VMEM((C, HID), jnp.float32),
            pltpu.SemaphoreType.DMA,
            pltpu.SemaphoreType.DMA,
        ],
    )
    return fn(a, b, dst3, src3)[0]


def _scatter_body(with_deg, m_hbm, dst_hbm, out_hbm, deg_hbm,
                  dst_v, rows0, rows1, zbuf, onesv, zdeg, acc_sh, deg_sh,
                  sem0, sem1):
    cid = lax.axis_index("c")
    sid = lax.axis_index("s")
    wid = sid * NC + cid
    base = wid * EW

    zero16 = jnp.zeros((L,), jnp.float32)

    def zrow(r, c2):
        for u in range(HID // L):
            zbuf[r, pl.ds(u * L, L)] = zero16
        return c2

    lax.fori_loop(0, NT, zrow, 0)
    pltpu.sync_copy(zbuf, acc_sh.at[pl.ds(sid * NT, NT)])

    if with_deg:
        one16 = jnp.ones((L,), jnp.float32)

        def orow(r, c2):
            onesv[r, :] = one16
            return c2

        lax.fori_loop(0, C, orow, 0)

        def zdrow(r, c2):
            zdeg[r, :] = zero16
            return c2

        lax.fori_loop(0, NT, zdrow, 0)
        pltpu.sync_copy(zdeg, deg_sh.at[pl.ds(sid * NT, NT)])

    plsc.subcore_barrier()

    pltpu.sync_copy(dst_hbm.at[wid], dst_v)

    bufs = ((rows0, sem0), (rows1, sem1))

    def start(j, rows, sem):
        pltpu.async_copy(m_hbm.at[pl.ds(base + j * C, C)], rows, sem)

    def process(j, rows, sem):
        pltpu.make_async_copy(m_hbm.at[pl.ds(0, C)], rows, sem).wait()
        pltpu.sync_copy(rows, acc_sh.at[dst_v.at[j]], add=True)
        if with_deg:
            pltpu.sync_copy(onesv, deg_sh.at[dst_v.at[j]], add=True)

    start(0, *bufs[0])
    start(1, *bufs[1])

    @pl.loop(0, K - 1, step=2)
    def _(g):
        for b in range(2):
            j = g + b
            rows, sem = bufs[b]
            process(j, rows, sem)

            @pl.when(j + 2 < K)
            def _():
                start(j + 2, rows, sem)

    process(K - 1, *bufs[(K - 1) % 2])

    plsc.subcore_barrier()

    sl = pl.ds(sid * NT, NT)
    pltpu.sync_copy(acc_sh.at[sl], zbuf)
    pltpu.sync_copy(zbuf, out_hbm.at[cid, sl])
    if with_deg:
        pltpu.sync_copy(deg_sh.at[sl], zdeg)
        pltpu.sync_copy(zdeg, deg_hbm.at[cid, sl])


def _scatter(m, dst3, with_deg):
    fn = pl.kernel(
        functools.partial(_scatter_body, with_deg),
        out_type=[_f32(NC, N2, HID), _f32(NC, N2, L)],
        mesh=_mesh,
        compiler_params=pltpu.CompilerParams(use_tc_tiling_on_sc=False),
        scratch_types=[
            pltpu.VMEM((K, C), jnp.int32),
            pltpu.VMEM((C, HID), jnp.float32),
            pltpu.VMEM((C, HID), jnp.float32),
            pltpu.VMEM((NT, HID), jnp.float32),
            pltpu.VMEM((C, L), jnp.float32),
            pltpu.VMEM((NT, L), jnp.float32),
            pltpu.VMEM_SHARED((N2, HID), jnp.float32),
            pltpu.VMEM_SHARED((N2, L), jnp.float32),
            pltpu.SemaphoreType.DMA,
            pltpu.SemaphoreType.DMA,
        ],
    )
    return fn(m, dst3)


def _proj_body(x_ref, wd_ref, wb_ref, b_ref, a_ref, bproj_ref):
    xv = x_ref[...]
    a_ref[...] = jnp.dot(xv, wd_ref[...],
                         preferred_element_type=jnp.float32) + b_ref[...]
    bproj_ref[...] = jnp.dot(xv, wb_ref[...],
                             preferred_element_type=jnp.float32)


def _proj(x, wd, wb, b):
    d = x.shape[1]
    bn = 2000
    grid = (N // bn,)
    return pl.pallas_call(
        _proj_body,
        grid=grid,
        in_specs=[
            pl.BlockSpec((bn, d), lambda i: (i, 0)),
            pl.BlockSpec((d, HID), lambda i: (0, 0)),
            pl.BlockSpec((d, HID), lambda i: (0, 0)),
            pl.BlockSpec((1, HID), lambda i: (0, 0)),
        ],
        out_specs=[
            pl.BlockSpec((bn, HID), lambda i: (i, 0)),
            pl.BlockSpec((bn, HID), lambda i: (i, 0)),
        ],
        out_shape=[_f32(N, HID), _f32(N, HID)],
    )(x, wd, wb, b.reshape(1, HID))


def _mlp_body(p_ref, w2_ref, b2_ref, w3_ref, b3_ref, m_ref):
    h = jnp.maximum(p_ref[...], 0.0)
    h = jnp.dot(h, w2_ref[...], preferred_element_type=jnp.float32) + b2_ref[...]
    h = jnp.maximum(h, 0.0)
    h = jnp.dot(h, w3_ref[...], preferred_element_type=jnp.float32) + b3_ref[...]
    m_ref[...] = jnp.maximum(h, 0.0)


def _edge_mlp(p, w2d, b2d, w3d, b3d):
    e2 = E // 2
    be = 4000
    p2 = p.reshape(e2, 2 * HID)
    out = pl.pallas_call(
        _mlp_body,
        grid=(e2 // be,),
        in_specs=[
            pl.BlockSpec((be, 2 * HID), lambda i: (i, 0)),
            pl.BlockSpec((2 * HID, 2 * HID), lambda i: (0, 0)),
            pl.BlockSpec((1, 2 * HID), lambda i: (0, 0)),
            pl.BlockSpec((2 * HID, 2 * HID), lambda i: (0, 0)),
            pl.BlockSpec((1, 2 * HID), lambda i: (0, 0)),
        ],
        out_specs=pl.BlockSpec((be, 2 * HID), lambda i: (i, 0)),
        out_shape=_f32(e2, 2 * HID),
    )(p2, w2d, b2d.reshape(1, 2 * HID), w3d, b3d.reshape(1, 2 * HID))
    return out.reshape(E, HID)


def _node1_body(s_ref, deg_ref, w4_ref, b4_ref, vd_ref, vb_ref, bv_ref,
                a_ref, b_ref):
    s = s_ref[0] + s_ref[1]
    deg = deg_ref[0, :, 0:1] + deg_ref[1, :, 0:1]
    agg = jnp.dot(s, w4_ref[...], preferred_element_type=jnp.float32)
    y = jnp.maximum(agg + deg * b4_ref[...], 0.0)
    a_ref[...] = jnp.dot(y, vd_ref[...],
                         preferred_element_type=jnp.float32) + bv_ref[...]
    b_ref[...] = jnp.dot(y, vb_ref[...], preferred_element_type=jnp.float32)


def _node1(s, deg, w4, b4, vd, vb, bv):
    bn = 2000
    return pl.pallas_call(
        _node1_body,
        grid=(N // bn,),
        in_specs=[
            pl.BlockSpec((NC, bn, HID), lambda i: (0, i, 0)),
            pl.BlockSpec((NC, bn, L), lambda i: (0, i, 0)),
            pl.BlockSpec((HID, HID), lambda i: (0, 0)),
            pl.BlockSpec((1, HID), lambda i: (0, 0)),
            pl.BlockSpec((HID, HID), lambda i: (0, 0)),
            pl.BlockSpec((HID, HID), lambda i: (0, 0)),
            pl.BlockSpec((1, HID), lambda i: (0, 0)),
        ],
        out_specs=[
            pl.BlockSpec((bn, HID), lambda i: (i, 0)),
            pl.BlockSpec((bn, HID), lambda i: (i, 0)),
        ],
        out_shape=[_f32(N, HID), _f32(N, HID)],
    )(s, deg, w4, b4.reshape(1, HID), vd, vb, bv.reshape(1, HID))


def _final_body(s_ref, deg_ref, w4_ref, b4_ref, o_ref):
    s = s_ref[0] + s_ref[1]
    deg = deg_ref[0, :, 0:1] + deg_ref[1, :, 0:1]
    agg = jnp.dot(s, w4_ref[...], preferred_element_type=jnp.float32)
    o_ref[...] = agg + deg * b4_ref[...]


def _final(s, deg, w4, b4):
    bn = 2000
    return pl.pallas_call(
        _final_body,
        grid=(N // bn,),
        in_specs=[
            pl.BlockSpec((NC, bn, HID), lambda i: (0, i, 0)),
            pl.BlockSpec((NC, bn, L), lambda i: (0, i, 0)),
            pl.BlockSpec((HID, D_OUT), lambda i: (0, 0)),
            pl.BlockSpec((1, D_OUT), lambda i: (0, 0)),
        ],
        out_specs=pl.BlockSpec((bn, D_OUT), lambda i: (i, 0)),
        out_shape=_f32(N, D_OUT),
    )(s, deg, w4, b4.reshape(1, D_OUT))


def _blockdiag(w):
    z = jnp.zeros_like(w)
    return jnp.concatenate(
        [jnp.concatenate([w, z], axis=1), jnp.concatenate([z, w], axis=1)],
        axis=0)


def kernel(x, edge_index, params):
    p1 = params["conv1"]
    p2 = params["conv2"]
    (w1, b1), (w2, b2), (w3, b3), (w4, b4) = p1
    (v1, c1), (v2, c2), (v3, c3), (v4, c4) = p2

    w1a, w1b = w1[:D_IN], w1[D_IN:]
    wd1 = w1a - w1b
    v1a, v1b = v1[:HID], v1[HID:]
    vd1 = v1a - v1b

    w2d, w3d = _blockdiag(w2), _blockdiag(w3)
    v2d, v3d = _blockdiag(v2), _blockdiag(v3)
    b2d = jnp.concatenate([b2, b2])
    b3d = jnp.concatenate([b3, b3])
    c2d = jnp.concatenate([c2, c2])
    c3d = jnp.concatenate([c3, c3])

    src3 = edge_index[0].reshape(NW, K, C)
    dst3 = edge_index[1].reshape(NW, K, C)

    a1, bp1 = _proj(x, wd1, w1b, b1)
    pre1 = _gather_add(a1, bp1, dst3, src3)
    m1 = _edge_mlp(pre1, w2d, b2d, w3d, b3d)
    s1, deg = _scatter(m1, dst3, with_deg=True)

    a2, bp2 = _node1(s1, deg, w4, b4, vd1, v1b, c1)
    pre2 = _gather_add(a2, bp2, dst3, src3)
    m2 = _edge_mlp(pre2, v2d, c2d, v3d, c3d)
    s2, _ = _scatter(m2, dst3, with_deg=False)

    return _final(s2, deg, v4, c4)
